# baseline (speedup 1.0000x reference)
import functools

import jax
import jax.numpy as jnp
from jax import lax
from jax.experimental import pallas as pl
from jax.experimental.pallas import tpu as pltpu, tpu_sc as plsc

N = 1024
D = 768

_info = plsc.get_sparse_core_info()
_NC = 1
_NS = _info.num_subcores
_NW = _NC * _NS
_RPW = N // _NW


@functools.partial(
    pl.kernel,
    mesh=plsc.VectorSubcoreMesh(
        core_axis_name="c", subcore_axis_name="s", num_cores=1),
    out_type=jax.ShapeDtypeStruct((N, D), jnp.float32),
    scratch_types=[pltpu.VMEM((_RPW, D), jnp.float32)],
)
def _pe_copy(pe_hbm, out_hbm, buf):
    wid = lax.axis_index("s") * _NC + lax.axis_index("c")
    base = wid * _RPW
    pltpu.sync_copy(pe_hbm.at[pl.ds(base, _RPW)], buf)
    pltpu.sync_copy(buf, out_hbm.at[pl.ds(base, _RPW)])


def kernel(h, w, pe):
    return _pe_copy(pe)[None]
